# Initial kernel scaffold; baseline (speedup 1.0000x reference)
import jax
import jax.numpy as jnp
from jax.experimental import pallas as pl


def kernel(x, edge_index, edge_attr, batch, n1_W1, n1_b1, n1_g, n1_be, n1_W2, n1_b2, root1, bias1, n2_W1, n2_b1, n2_g, n2_be, n2_W2, n2_b2, root2, bias2, fc_W, fc_b):
    raise NotImplementedError("write your pallas kernel here")


# speedup vs baseline: 1.1453x; 1.1453x over previous
import functools

import jax
import jax.numpy as jnp
import numpy as np
from jax import lax
from jax.experimental import pallas as pl
from jax.experimental.pallas import tpu as pltpu
from jax.experimental.pallas import tpu_sc as plsc

N = 10000
E = 160000
DIN = 128
DH = 16
DOUT = 10
G = 8
EPS = 1e-5
K = 25
KP = 32
PW = (K + 1) * DH

NC = 2
NS = 16
NW = NC * NS
EP = 163840
EPW = EP // NW
B = 64
NB = EPW // B
GPB = B // 16
ZPS = 640
ZR = 128
NTRASH = NS * ZPS
DS = 624
DOFF = NS * DS
DREM = N - DOFF

BE = 4096
BN = 2000


def _c16(v):
    return jnp.full((16,), v, jnp.int32)


def _edge_mlp_body(ea_ref, w1_ref, b1_ref, w2_ref, b2_ref, r1_ref, r2_ref):
    ea = ea_ref[...]
    for w_ref, b_ref, out_ref in ((w1_ref, b1_ref, r1_ref), (w2_ref, b2_ref, r2_ref)):
        w = w_ref[...]
        acc = b_ref[...]
        acc = acc + ea[:, 0:1] * w[0:1, :]
        acc = acc + ea[:, 1:2] * w[1:2, :]
        acc = acc + ea[:, 2:3] * w[2:3, :]
        out_ref[...] = jnp.maximum(acc, 0.0)


def _edge_mlp(ea, w1a, b1a, w1b, b1b):
    return pl.pallas_call(
        _edge_mlp_body,
        grid=(EP // BE,),
        in_specs=[
            pl.BlockSpec((BE, 3), lambda i: (i, 0)),
            pl.BlockSpec((3, KP), lambda i: (0, 0)),
            pl.BlockSpec((1, KP), lambda i: (0, 0)),
            pl.BlockSpec((3, KP), lambda i: (0, 0)),
            pl.BlockSpec((1, KP), lambda i: (0, 0)),
        ],
        out_specs=[
            pl.BlockSpec((BE, KP), lambda i: (i, 0)),
            pl.BlockSpec((BE, KP), lambda i: (i, 0)),
        ],
        out_shape=[
            jax.ShapeDtypeStruct((EP, KP), jnp.float32),
            jax.ShapeDtypeStruct((EP, KP), jnp.float32),
        ],
    )(ea, w1a, b1a, w1b, b1b)


def _node_table_body(x_ref, wb_ref, root_ref, p_ref, xr_ref):
    x = x_ref[...]
    p_ref[...] = jnp.dot(x, wb_ref[...], preferred_element_type=jnp.float32)
    xr_ref[...] = jnp.dot(x, root_ref[...], preferred_element_type=jnp.float32)


def _node_table(x, wbig, root):
    cin = x.shape[1]
    return pl.pallas_call(
        _node_table_body,
        grid=(N // BN,),
        in_specs=[
            pl.BlockSpec((BN, cin), lambda i: (i, 0)),
            pl.BlockSpec((cin, PW), lambda i: (0, 0)),
            pl.BlockSpec((cin, DH), lambda i: (0, 0)),
        ],
        out_specs=[
            pl.BlockSpec((BN, PW), lambda i: (i, 0)),
            pl.BlockSpec((BN, DH), lambda i: (i, 0)),
        ],
        out_shape=[
            jax.ShapeDtypeStruct((N, PW), jnp.float32),
            jax.ShapeDtypeStruct((N, DH), jnp.float32),
        ],
    )(x, wbig, root)


def _edge_pass_body(with_deg, p_hbm, r_hbm, src_hbm, dst_hbm, *rest):
    if with_deg:
        acc_out, deg_out = rest[0], rest[1]
        scratch = rest[2:]
    else:
        acc_out = rest[0]
        deg_out = None
        scratch = rest[1:]
    (src_v, dst_v, r_v, rows0, rows1, msg_v, ones_v, zbuf,
     acc_sh, deg_sh, sem0, sem1) = scratch
    rows = (rows0, rows1)
    sems = (sem0, sem1)

    c = lax.axis_index("c")
    s = lax.axis_index("s")
    wid = s * NC + c

    pltpu.sync_copy(src_hbm.at[wid], src_v)
    pltpu.sync_copy(dst_hbm.at[wid], dst_v)

    def zrow(i, _):
        zbuf[i, :] = jnp.zeros((DH,), jnp.float32)
        return 0
    lax.fori_loop(0, ZR, zrow, 0)

    def onesrow(i, _):
        ones_v[i, :] = jnp.ones((DH,), jnp.float32)
        return 0
    lax.fori_loop(0, B, onesrow, 0)

    def zcp(j, _):
        pltpu.sync_copy(zbuf, acc_sh.at[pl.ds(s * ZPS + j * ZR, ZR)])
        if with_deg:
            pltpu.sync_copy(zbuf, deg_sh.at[pl.ds(s * ZPS + j * ZR, ZR)])
        return 0
    lax.fori_loop(0, ZPS // ZR, zcp, 0)

    plsc.subcore_barrier()

    def issue(b, par):
        pltpu.async_copy(p_hbm.at[src_v.at[b]], rows[par], sems[par])

    def drain(par):
        pltpu.make_async_copy(p_hbm.at[pl.ds(0, B)], rows[par], sems[par]).wait()

    issue(0, 0)
    issue(1, 1)

    def pair_body(jj, _):
        j0 = jj * 2
        for par in range(2):
            b = j0 + par
            rbuf = rows[par]
            pltpu.sync_copy(r_hbm.at[pl.ds(wid * EPW + b * B, B)], r_v)
            drain(par)

            def group_body(g_, _g):
                le = lax.iota(jnp.int32, 16) + g_ * 16
                rk = [plsc.load_gather(r_v, [le, _c16(k)]) for k in range(K)]
                for o in range(DH):
                    acc = plsc.load_gather(rbuf, [le, _c16(K * DH + o)])
                    for k in range(K):
                        acc = acc + rk[k] * plsc.load_gather(
                            rbuf, [le, _c16(k * DH + o)])
                    plsc.store_scatter(msg_v, [le, _c16(o)], acc)
                return 0
            lax.fori_loop(0, GPB, group_body, 0)

            pltpu.sync_copy(msg_v, acc_sh.at[dst_v.at[b]], add=True)
            if with_deg:
                pltpu.sync_copy(ones_v, deg_sh.at[dst_v.at[b]], add=True)

            @pl.when(b + 2 < NB)
            def _():
                issue(b + 2, par)
        return 0
    lax.fori_loop(0, NB // 2, pair_body, 0)

    plsc.subcore_barrier()

    pltpu.sync_copy(acc_sh.at[pl.ds(s * DS, DS)],
                    acc_out.at[c, pl.ds(s * DS, DS)])
    if with_deg:
        pltpu.sync_copy(deg_sh.at[pl.ds(s * DS, DS)],
                        deg_out.at[c, pl.ds(s * DS, DS)])

    @pl.when(s == NS - 1)
    def _():
        pltpu.sync_copy(acc_sh.at[pl.ds(DOFF, DREM)],
                        acc_out.at[c, pl.ds(DOFF, DREM)])
        if with_deg:
            pltpu.sync_copy(deg_sh.at[pl.ds(DOFF, DREM)],
                            deg_out.at[c, pl.ds(DOFF, DREM)])


def _edge_pass(p_tab, r_tab, srcr, dstr, with_deg):
    out_type = [jax.ShapeDtypeStruct((NC, N, DH), jnp.float32)]
    if with_deg:
        out_type.append(jax.ShapeDtypeStruct((NC, N, DH), jnp.float32))
    mesh = plsc.VectorSubcoreMesh(core_axis_name="c", subcore_axis_name="s")
    fn = pl.kernel(
        functools.partial(_edge_pass_body, with_deg),
        out_type=out_type,
        mesh=mesh,
        compiler_params=pltpu.CompilerParams(needs_layout_passes=False,
                                             use_tc_tiling_on_sc=False),
        scratch_types=[
            pltpu.VMEM((NB, B), jnp.int32),
            pltpu.VMEM((NB, B), jnp.int32),
            pltpu.VMEM((B, KP), jnp.float32),
            pltpu.VMEM((B, PW), jnp.float32),
            pltpu.VMEM((B, PW), jnp.float32),
            pltpu.VMEM((B, DH), jnp.float32),
            pltpu.VMEM((B, DH), jnp.float32),
            pltpu.VMEM((ZR, DH), jnp.float32),
            pltpu.VMEM_SHARED((NTRASH, DH), jnp.float32),
            pltpu.VMEM_SHARED((NTRASH, DH), jnp.float32),
            pltpu.SemaphoreType.DMA,
            pltpu.SemaphoreType.DMA,
        ],
    )
    return fn(p_tab, r_tab, srcr, dstr)


def _combine_body(a0_ref, a1_ref, d0_ref, d1_ref, xr_ref, bias_ref,
                  wb_ref, root_ref, p_ref, hr_ref):
    deg = jnp.maximum(d0_ref[...] + d1_ref[...], 1.0)
    agg = (a0_ref[...] + a1_ref[...]) / deg
    z = agg + xr_ref[...] + bias_ref[...]
    h = jnp.where(z > 0, z, jnp.exp(jnp.minimum(z, 0.0)) - 1.0)
    p_ref[...] = jnp.dot(h, wb_ref[...], preferred_element_type=jnp.float32)
    hr_ref[...] = jnp.dot(h, root_ref[...], preferred_element_type=jnp.float32)


def _combine(a0, a1, d0, d1, xr, bias, wbig, root):
    return pl.pallas_call(
        _combine_body,
        grid=(N // BN,),
        in_specs=[
            pl.BlockSpec((BN, DH), lambda i: (i, 0)),
            pl.BlockSpec((BN, DH), lambda i: (i, 0)),
            pl.BlockSpec((BN, DH), lambda i: (i, 0)),
            pl.BlockSpec((BN, DH), lambda i: (i, 0)),
            pl.BlockSpec((BN, DH), lambda i: (i, 0)),
            pl.BlockSpec((1, DH), lambda i: (0, 0)),
            pl.BlockSpec((DH, PW), lambda i: (0, 0)),
            pl.BlockSpec((DH, DH), lambda i: (0, 0)),
        ],
        out_specs=[
            pl.BlockSpec((BN, PW), lambda i: (i, 0)),
            pl.BlockSpec((BN, DH), lambda i: (i, 0)),
        ],
        out_shape=[
            jax.ShapeDtypeStruct((N, PW), jnp.float32),
            jax.ShapeDtypeStruct((N, DH), jnp.float32),
        ],
    )(a0, a1, d0, d1, xr, bias, wbig, root)


def _final_body(a0_ref, a1_ref, d0_ref, d1_ref, hr_ref, bias_ref, batch_ref,
                fcw_ref, fcb_ref, psum_ref, csum_ref, out_ref):
    step = pl.program_id(0)

    @pl.when(step == 0)
    def _():
        psum_ref[...] = jnp.zeros_like(psum_ref)
        csum_ref[...] = jnp.zeros_like(csum_ref)

    deg = jnp.maximum(d0_ref[...] + d1_ref[...], 1.0)
    agg = (a0_ref[...] + a1_ref[...]) / deg
    z = agg + hr_ref[...] + bias_ref[...]
    h = jnp.where(z > 0, z, jnp.exp(jnp.minimum(z, 0.0)) - 1.0)

    row = batch_ref[...].reshape(1, BN)
    oh = (lax.broadcasted_iota(jnp.int32, (G, BN), 0)
          == jnp.broadcast_to(row, (G, BN))).astype(jnp.float32)
    psum_ref[...] += jnp.dot(oh, h, preferred_element_type=jnp.float32)
    csum_ref[...] += jnp.sum(oh, axis=1, keepdims=True) * jnp.ones(
        (1, DH), jnp.float32)

    @pl.when(step == (N // BN) - 1)
    def _():
        pooled = psum_ref[...] / jnp.maximum(csum_ref[...], 1.0)
        out_ref[...] = jnp.dot(pooled, fcw_ref[...],
                               preferred_element_type=jnp.float32) + fcb_ref[...]


def _final(a0, a1, d0, d1, hr, bias, batch3, fcw, fcb):
    outs = pl.pallas_call(
        _final_body,
        grid=(N // BN,),
        in_specs=[
            pl.BlockSpec((BN, DH), lambda i: (i, 0)),
            pl.BlockSpec((BN, DH), lambda i: (i, 0)),
            pl.BlockSpec((BN, DH), lambda i: (i, 0)),
            pl.BlockSpec((BN, DH), lambda i: (i, 0)),
            pl.BlockSpec((BN, DH), lambda i: (i, 0)),
            pl.BlockSpec((1, DH), lambda i: (0, 0)),
            pl.BlockSpec((1, 1, BN), lambda i: (i, 0, 0)),
            pl.BlockSpec((DH, DOUT), lambda i: (0, 0)),
            pl.BlockSpec((1, DOUT), lambda i: (0, 0)),
        ],
        out_specs=[
            pl.BlockSpec((G, DH), lambda i: (0, 0)),
            pl.BlockSpec((G, DH), lambda i: (0, 0)),
            pl.BlockSpec((G, DOUT), lambda i: (0, 0)),
        ],
        out_shape=[
            jax.ShapeDtypeStruct((G, DH), jnp.float32),
            jax.ShapeDtypeStruct((G, DH), jnp.float32),
            jax.ShapeDtypeStruct((G, DOUT), jnp.float32),
        ],
    )(a0, a1, d0, d1, hr, bias, batch3, fcw, fcb)
    return outs[2]


def _big_w(w2, b2, cin):
    a = w2.reshape(K, cin, DH).transpose(1, 0, 2).reshape(cin, K * DH)
    return jnp.concatenate([a, b2.reshape(cin, DH)], axis=1)


def _pad_mlp_params(w1, b1, g, be):
    s = jnp.float32(1.0) / jnp.sqrt(jnp.float32(1.0 + EPS))
    ge = g * s
    wa = jnp.pad(w1 * ge, ((0, 0), (0, KP - K)))
    ba = jnp.pad((b1 * ge + be).reshape(1, K), ((0, 0), (0, KP - K)))
    return wa, ba


def kernel(x, edge_index, edge_attr, batch, n1_W1, n1_b1, n1_g, n1_be, n1_W2,
           n1_b2, root1, bias1, n2_W1, n2_b1, n2_g, n2_be, n2_W2, n2_b2,
           root2, bias2, fc_W, fc_b):
    w1a, b1a = _pad_mlp_params(n1_W1, n1_b1, n1_g, n1_be)
    w1b, b1b = _pad_mlp_params(n2_W1, n2_b1, n2_g, n2_be)

    wb1 = _big_w(n1_W2, n1_b2, DIN)
    wb2 = _big_w(n2_W2, n2_b2, DH)

    eap = jnp.pad(edge_attr, ((0, EP - E), (0, 0)))
    srcr = jnp.pad(edge_index[0], (0, EP - E)).reshape(NW, NB, B)
    dstr = jnp.pad(edge_index[1], (0, EP - E),
                   constant_values=N).reshape(NW, NB, B)
    batch3 = batch.reshape(N // BN, 1, BN)

    r1, r2 = _edge_mlp(eap, w1a, b1a, w1b, b1b)
    p1, xr1 = _node_table(x, wb1, root1)
    acc1, deg = _edge_pass(p1, r1, srcr, dstr, with_deg=True)
    p2, hr2 = _combine(acc1[0], acc1[1], deg[0], deg[1], xr1,
                       bias1.reshape(1, DH), wb2, root2)
    (acc2,) = _edge_pass(p2, r2, srcr, dstr, with_deg=False)
    out = _final(acc2[0], acc2[1], deg[0], deg[1], hr2,
                 bias2.reshape(1, DH), batch3, fc_W, fc_b.reshape(1, DOUT))
    return out


# speedup vs baseline: 1.1950x; 1.0434x over previous
import functools

import jax
import jax.numpy as jnp
import numpy as np
from jax import lax
from jax.experimental import pallas as pl
from jax.experimental.pallas import tpu as pltpu
from jax.experimental.pallas import tpu_sc as plsc

N = 10000
E = 160000
DIN = 128
DH = 16
DOUT = 10
G = 8
EPS = 1e-5
K = 25
KP = 32
PW = (K + 1) * DH

NC = 2
NS = 16
NW = NC * NS
EP = 163840
EPW = EP // NW
B = 32
NB = EPW // B
GPB = B // 16
NBUF = 4
ZPS = 640
ZR = 128
NTRASH = NS * ZPS
DS = 624
DOFF = NS * DS
DREM = N - DOFF

BE = 4096
BN = 2000


def _c16(v):
    return jnp.full((16,), v, jnp.int32)


def _edge_mlp_body(ea_ref, w1_ref, b1_ref, w2_ref, b2_ref, r1_ref, r2_ref):
    ea = ea_ref[...]
    for w_ref, b_ref, out_ref in ((w1_ref, b1_ref, r1_ref), (w2_ref, b2_ref, r2_ref)):
        w = w_ref[...]
        acc = b_ref[...]
        acc = acc + ea[:, 0:1] * w[0:1, :]
        acc = acc + ea[:, 1:2] * w[1:2, :]
        acc = acc + ea[:, 2:3] * w[2:3, :]
        out_ref[...] = jnp.maximum(acc, 0.0)


def _edge_mlp(ea, w1a, b1a, w1b, b1b):
    return pl.pallas_call(
        _edge_mlp_body,
        grid=(EP // BE,),
        in_specs=[
            pl.BlockSpec((BE, 3), lambda i: (i, 0)),
            pl.BlockSpec((3, KP), lambda i: (0, 0)),
            pl.BlockSpec((1, KP), lambda i: (0, 0)),
            pl.BlockSpec((3, KP), lambda i: (0, 0)),
            pl.BlockSpec((1, KP), lambda i: (0, 0)),
        ],
        out_specs=[
            pl.BlockSpec((BE, KP), lambda i: (i, 0)),
            pl.BlockSpec((BE, KP), lambda i: (i, 0)),
        ],
        out_shape=[
            jax.ShapeDtypeStruct((EP, KP), jnp.float32),
            jax.ShapeDtypeStruct((EP, KP), jnp.float32),
        ],
    )(ea, w1a, b1a, w1b, b1b)


def _node_table_body(x_ref, wb_ref, root_ref, p_ref, xr_ref):
    x = x_ref[...]
    p_ref[...] = jnp.dot(x, wb_ref[...], preferred_element_type=jnp.float32)
    xr_ref[...] = jnp.dot(x, root_ref[...], preferred_element_type=jnp.float32)


def _node_table(x, wbig, root):
    cin = x.shape[1]
    return pl.pallas_call(
        _node_table_body,
        grid=(N // BN,),
        in_specs=[
            pl.BlockSpec((BN, cin), lambda i: (i, 0)),
            pl.BlockSpec((cin, PW), lambda i: (0, 0)),
            pl.BlockSpec((cin, DH), lambda i: (0, 0)),
        ],
        out_specs=[
            pl.BlockSpec((BN, PW), lambda i: (i, 0)),
            pl.BlockSpec((BN, DH), lambda i: (i, 0)),
        ],
        out_shape=[
            jax.ShapeDtypeStruct((N, PW), jnp.float32),
            jax.ShapeDtypeStruct((N, DH), jnp.float32),
        ],
    )(x, wbig, root)


def _edge_pass_body(with_deg, p_hbm, r_hbm, src_hbm, dst_hbm, *rest):
    if with_deg:
        acc_out, deg_out = rest[0], rest[1]
        scratch = rest[2:]
    else:
        acc_out = rest[0]
        deg_out = None
        scratch = rest[1:]
    src_v, dst_v = scratch[0], scratch[1]
    r_v = scratch[2:2 + NBUF]
    rows = scratch[2 + NBUF:2 + 2 * NBUF]
    msg_v, ones_v, zbuf = scratch[2 + 2 * NBUF:5 + 2 * NBUF]
    acc_sh, deg_sh = scratch[5 + 2 * NBUF], scratch[6 + 2 * NBUF]
    gsem = scratch[7 + 2 * NBUF:7 + 3 * NBUF]
    rsem = scratch[7 + 3 * NBUF:7 + 4 * NBUF]

    c = lax.axis_index("c")
    s = lax.axis_index("s")
    wid = s * NC + c

    pltpu.sync_copy(src_hbm.at[wid], src_v)
    pltpu.sync_copy(dst_hbm.at[wid], dst_v)

    def zrow(i, _):
        zbuf[i, :] = jnp.zeros((DH,), jnp.float32)
        return 0
    lax.fori_loop(0, ZR, zrow, 0)

    def onesrow(i, _):
        ones_v[i, :] = jnp.ones((DH,), jnp.float32)
        return 0
    lax.fori_loop(0, B, onesrow, 0)

    def zcp(j, _):
        pltpu.sync_copy(zbuf, acc_sh.at[pl.ds(s * ZPS + j * ZR, ZR)])
        if with_deg:
            pltpu.sync_copy(zbuf, deg_sh.at[pl.ds(s * ZPS + j * ZR, ZR)])
        return 0
    lax.fori_loop(0, ZPS // ZR, zcp, 0)

    plsc.subcore_barrier()

    def issue(b, par):
        pltpu.async_copy(p_hbm.at[src_v.at[b]], rows[par], gsem[par])
        pltpu.async_copy(r_hbm.at[pl.ds(wid * EPW + b * B, B)], r_v[par],
                         rsem[par])

    def drain(par):
        pltpu.make_async_copy(p_hbm.at[pl.ds(0, B)], rows[par],
                              gsem[par]).wait()
        pltpu.make_async_copy(r_hbm.at[pl.ds(0, B)], r_v[par],
                              rsem[par]).wait()

    for par in range(NBUF):
        issue(par, par)

    def ring_body(jj, _):
        j0 = jj * NBUF
        for par in range(NBUF):
            b = j0 + par
            rbuf = rows[par]
            rcoef = r_v[par]
            drain(par)

            def group_body(g_, _g):
                le = lax.iota(jnp.int32, 16) + g_ * 16
                rk = [plsc.load_gather(rcoef, [le, _c16(k)]) for k in range(K)]
                for o in range(DH):
                    acc = plsc.load_gather(rbuf, [le, _c16(K * DH + o)])
                    for k in range(K):
                        acc = acc + rk[k] * plsc.load_gather(
                            rbuf, [le, _c16(k * DH + o)])
                    plsc.store_scatter(msg_v, [le, _c16(o)], acc)
                return 0
            lax.fori_loop(0, GPB, group_body, 0)

            pltpu.sync_copy(msg_v, acc_sh.at[dst_v.at[b]], add=True)
            if with_deg:
                pltpu.sync_copy(ones_v, deg_sh.at[dst_v.at[b]], add=True)

            @pl.when(b + NBUF < NB)
            def _():
                issue(b + NBUF, par)
        return 0
    lax.fori_loop(0, NB // NBUF, ring_body, 0)

    plsc.subcore_barrier()

    pltpu.sync_copy(acc_sh.at[pl.ds(s * DS, DS)],
                    acc_out.at[c, pl.ds(s * DS, DS)])
    if with_deg:
        pltpu.sync_copy(deg_sh.at[pl.ds(s * DS, DS)],
                        deg_out.at[c, pl.ds(s * DS, DS)])

    @pl.when(s == NS - 1)
    def _():
        pltpu.sync_copy(acc_sh.at[pl.ds(DOFF, DREM)],
                        acc_out.at[c, pl.ds(DOFF, DREM)])
        if with_deg:
            pltpu.sync_copy(deg_sh.at[pl.ds(DOFF, DREM)],
                            deg_out.at[c, pl.ds(DOFF, DREM)])


def _edge_pass(p_tab, r_tab, srcr, dstr, with_deg):
    out_type = [jax.ShapeDtypeStruct((NC, N, DH), jnp.float32)]
    if with_deg:
        out_type.append(jax.ShapeDtypeStruct((NC, N, DH), jnp.float32))
    mesh = plsc.VectorSubcoreMesh(core_axis_name="c", subcore_axis_name="s")
    fn = pl.kernel(
        functools.partial(_edge_pass_body, with_deg),
        out_type=out_type,
        mesh=mesh,
        compiler_params=pltpu.CompilerParams(needs_layout_passes=False,
                                             use_tc_tiling_on_sc=False),
        scratch_types=(
            [pltpu.VMEM((NB, B), jnp.int32)] * 2
            + [pltpu.VMEM((B, KP), jnp.float32)] * NBUF
            + [pltpu.VMEM((B, PW), jnp.float32)] * NBUF
            + [pltpu.VMEM((B, DH), jnp.float32)] * 2
            + [pltpu.VMEM((ZR, DH), jnp.float32)]
            + [pltpu.VMEM_SHARED((NTRASH, DH), jnp.float32)] * 2
            + [pltpu.SemaphoreType.DMA] * (2 * NBUF)
        ),
    )
    return fn(p_tab, r_tab, srcr, dstr)


def _combine_body(a0_ref, a1_ref, d0_ref, d1_ref, xr_ref, bias_ref,
                  wb_ref, root_ref, p_ref, hr_ref):
    deg = jnp.maximum(d0_ref[...] + d1_ref[...], 1.0)
    agg = (a0_ref[...] + a1_ref[...]) / deg
    z = agg + xr_ref[...] + bias_ref[...]
    h = jnp.where(z > 0, z, jnp.exp(jnp.minimum(z, 0.0)) - 1.0)
    p_ref[...] = jnp.dot(h, wb_ref[...], preferred_element_type=jnp.float32)
    hr_ref[...] = jnp.dot(h, root_ref[...], preferred_element_type=jnp.float32)


def _combine(a0, a1, d0, d1, xr, bias, wbig, root):
    return pl.pallas_call(
        _combine_body,
        grid=(N // BN,),
        in_specs=[
            pl.BlockSpec((BN, DH), lambda i: (i, 0)),
            pl.BlockSpec((BN, DH), lambda i: (i, 0)),
            pl.BlockSpec((BN, DH), lambda i: (i, 0)),
            pl.BlockSpec((BN, DH), lambda i: (i, 0)),
            pl.BlockSpec((BN, DH), lambda i: (i, 0)),
            pl.BlockSpec((1, DH), lambda i: (0, 0)),
            pl.BlockSpec((DH, PW), lambda i: (0, 0)),
            pl.BlockSpec((DH, DH), lambda i: (0, 0)),
        ],
        out_specs=[
            pl.BlockSpec((BN, PW), lambda i: (i, 0)),
            pl.BlockSpec((BN, DH), lambda i: (i, 0)),
        ],
        out_shape=[
            jax.ShapeDtypeStruct((N, PW), jnp.float32),
            jax.ShapeDtypeStruct((N, DH), jnp.float32),
        ],
    )(a0, a1, d0, d1, xr, bias, wbig, root)


def _final_body(a0_ref, a1_ref, d0_ref, d1_ref, hr_ref, bias_ref, batch_ref,
                fcw_ref, fcb_ref, psum_ref, csum_ref, out_ref):
    step = pl.program_id(0)

    @pl.when(step == 0)
    def _():
        psum_ref[...] = jnp.zeros_like(psum_ref)
        csum_ref[...] = jnp.zeros_like(csum_ref)

    deg = jnp.maximum(d0_ref[...] + d1_ref[...], 1.0)
    agg = (a0_ref[...] + a1_ref[...]) / deg
    z = agg + hr_ref[...] + bias_ref[...]
    h = jnp.where(z > 0, z, jnp.exp(jnp.minimum(z, 0.0)) - 1.0)

    row = batch_ref[...].reshape(1, BN)
    oh = (lax.broadcasted_iota(jnp.int32, (G, BN), 0)
          == jnp.broadcast_to(row, (G, BN))).astype(jnp.float32)
    psum_ref[...] += jnp.dot(oh, h, preferred_element_type=jnp.float32)
    csum_ref[...] += jnp.sum(oh, axis=1, keepdims=True) * jnp.ones(
        (1, DH), jnp.float32)

    @pl.when(step == (N // BN) - 1)
    def _():
        pooled = psum_ref[...] / jnp.maximum(csum_ref[...], 1.0)
        out_ref[...] = jnp.dot(pooled, fcw_ref[...],
                               preferred_element_type=jnp.float32) + fcb_ref[...]


def _final(a0, a1, d0, d1, hr, bias, batch3, fcw, fcb):
    outs = pl.pallas_call(
        _final_body,
        grid=(N // BN,),
        in_specs=[
            pl.BlockSpec((BN, DH), lambda i: (i, 0)),
            pl.BlockSpec((BN, DH), lambda i: (i, 0)),
            pl.BlockSpec((BN, DH), lambda i: (i, 0)),
            pl.BlockSpec((BN, DH), lambda i: (i, 0)),
            pl.BlockSpec((BN, DH), lambda i: (i, 0)),
            pl.BlockSpec((1, DH), lambda i: (0, 0)),
            pl.BlockSpec((1, 1, BN), lambda i: (i, 0, 0)),
            pl.BlockSpec((DH, DOUT), lambda i: (0, 0)),
            pl.BlockSpec((1, DOUT), lambda i: (0, 0)),
        ],
        out_specs=[
            pl.BlockSpec((G, DH), lambda i: (0, 0)),
            pl.BlockSpec((G, DH), lambda i: (0, 0)),
            pl.BlockSpec((G, DOUT), lambda i: (0, 0)),
        ],
        out_shape=[
            jax.ShapeDtypeStruct((G, DH), jnp.float32),
            jax.ShapeDtypeStruct((G, DH), jnp.float32),
            jax.ShapeDtypeStruct((G, DOUT), jnp.float32),
        ],
    )(a0, a1, d0, d1, hr, bias, batch3, fcw, fcb)
    return outs[2]


def _big_w(w2, b2, cin):
    a = w2.reshape(K, cin, DH).transpose(1, 0, 2).reshape(cin, K * DH)
    return jnp.concatenate([a, b2.reshape(cin, DH)], axis=1)


def _pad_mlp_params(w1, b1, g, be):
    s = jnp.float32(1.0) / jnp.sqrt(jnp.float32(1.0 + EPS))
    ge = g * s
    wa = jnp.pad(w1 * ge, ((0, 0), (0, KP - K)))
    ba = jnp.pad((b1 * ge + be).reshape(1, K), ((0, 0), (0, KP - K)))
    return wa, ba


def kernel(x, edge_index, edge_attr, batch, n1_W1, n1_b1, n1_g, n1_be, n1_W2,
           n1_b2, root1, bias1, n2_W1, n2_b1, n2_g, n2_be, n2_W2, n2_b2,
           root2, bias2, fc_W, fc_b):
    w1a, b1a = _pad_mlp_params(n1_W1, n1_b1, n1_g, n1_be)
    w1b, b1b = _pad_mlp_params(n2_W1, n2_b1, n2_g, n2_be)

    wb1 = _big_w(n1_W2, n1_b2, DIN)
    wb2 = _big_w(n2_W2, n2_b2, DH)

    eap = jnp.pad(edge_attr, ((0, EP - E), (0, 0)))
    srcr = jnp.pad(edge_index[0], (0, EP - E)).reshape(NW, NB, B)
    dstr = jnp.pad(edge_index[1], (0, EP - E),
                   constant_values=N).reshape(NW, NB, B)
    batch3 = batch.reshape(N // BN, 1, BN)

    r1, r2 = _edge_mlp(eap, w1a, b1a, w1b, b1b)
    p1, xr1 = _node_table(x, wb1, root1)
    acc1, deg = _edge_pass(p1, r1, srcr, dstr, with_deg=True)
    p2, hr2 = _combine(acc1[0], acc1[1], deg[0], deg[1], xr1,
                       bias1.reshape(1, DH), wb2, root2)
    (acc2,) = _edge_pass(p2, r2, srcr, dstr, with_deg=False)
    out = _final(acc2[0], acc2[1], deg[0], deg[1], hr2,
                 bias2.reshape(1, DH), batch3, fc_W, fc_b.reshape(1, DOUT))
    return out


# speedup vs baseline: 2.1057x; 1.7620x over previous
import functools

import jax
import jax.numpy as jnp
import numpy as np
from jax import lax
from jax.experimental import pallas as pl
from jax.experimental.pallas import tpu as pltpu
from jax.experimental.pallas import tpu_sc as plsc

N = 10000
E = 160000
DIN = 128
DH = 16
DOUT = 10
G = 8
EPS = 1e-5
K = 25
KP = 32
PW = (K + 1) * DH
PB = PW // 2
WPB = DH // 2

NC = 2
NS = 16
NW = NC * NS
EP = 163840
EPW = EP // NW
B = 64
NB = EPW // B
GPB = B // 16
NBUF = 4
ZPS = 640
ZR = 64
NTRASH = NS * ZPS
DS = 624
DOFF = NS * DS
DREM = N - DOFF

BE = 4096
BN = 2000


def _c16(v):
    return jnp.full((16,), v, jnp.int32)


def _edge_mlp_body(ea_ref, w1_ref, b1_ref, w2_ref, b2_ref, r1_ref, r2_ref):
    ea = ea_ref[...]
    for w_ref, b_ref, out_ref in ((w1_ref, b1_ref, r1_ref), (w2_ref, b2_ref, r2_ref)):
        w = w_ref[...]
        acc = b_ref[...]
        acc = acc + ea[:, 0:1] * w[0:1, :]
        acc = acc + ea[:, 1:2] * w[1:2, :]
        acc = acc + ea[:, 2:3] * w[2:3, :]
        out_ref[...] = jnp.maximum(acc, 0.0)


def _edge_mlp(ea, w1a, b1a, w1b, b1b):
    return pl.pallas_call(
        _edge_mlp_body,
        grid=(EP // BE,),
        in_specs=[
            pl.BlockSpec((BE, 3), lambda i: (i, 0)),
            pl.BlockSpec((3, KP), lambda i: (0, 0)),
            pl.BlockSpec((1, KP), lambda i: (0, 0)),
            pl.BlockSpec((3, KP), lambda i: (0, 0)),
            pl.BlockSpec((1, KP), lambda i: (0, 0)),
        ],
        out_specs=[
            pl.BlockSpec((BE, KP), lambda i: (i, 0)),
            pl.BlockSpec((BE, KP), lambda i: (i, 0)),
        ],
        out_shape=[
            jax.ShapeDtypeStruct((EP, KP), jnp.float32),
            jax.ShapeDtypeStruct((EP, KP), jnp.float32),
        ],
    )(ea, w1a, b1a, w1b, b1b)


def _node_table_body(x_ref, wb_ref, root_ref, p_ref, xr_ref):
    x = x_ref[...]
    p_ref[...] = jnp.dot(x, wb_ref[...],
                         preferred_element_type=jnp.float32).astype(jnp.bfloat16)
    xr_ref[...] = jnp.dot(x, root_ref[...], preferred_element_type=jnp.float32)


def _node_table(x, wbig, root):
    cin = x.shape[1]
    return pl.pallas_call(
        _node_table_body,
        grid=(N // BN,),
        in_specs=[
            pl.BlockSpec((BN, cin), lambda i: (i, 0)),
            pl.BlockSpec((cin, PW), lambda i: (0, 0)),
            pl.BlockSpec((cin, DH), lambda i: (0, 0)),
        ],
        out_specs=[
            pl.BlockSpec((BN, PW), lambda i: (i, 0)),
            pl.BlockSpec((BN, DH), lambda i: (i, 0)),
        ],
        out_shape=[
            jax.ShapeDtypeStruct((N, PW), jnp.bfloat16),
            jax.ShapeDtypeStruct((N, DH), jnp.float32),
        ],
    )(x, wbig, root)


def _edge_pass_body(with_deg, p_hbm, r_hbm, src_hbm, dst_hbm, *rest):
    if with_deg:
        acc_out, deg_out = rest[0], rest[1]
        scratch = rest[2:]
    else:
        acc_out = rest[0]
        deg_out = None
        scratch = rest[1:]
    src_v, dst_v = scratch[0], scratch[1]
    r_v = scratch[2:2 + NBUF]
    rows = scratch[2 + NBUF:2 + 2 * NBUF]
    msg_v, ones_v, zbuf = scratch[2 + 2 * NBUF:5 + 2 * NBUF]
    acc_sh, deg_sh = scratch[5 + 2 * NBUF], scratch[6 + 2 * NBUF]
    gsem = scratch[7 + 2 * NBUF:7 + 3 * NBUF]
    rsem = scratch[7 + 3 * NBUF:7 + 4 * NBUF]

    c = lax.axis_index("c")
    s = lax.axis_index("s")
    wid = s * NC + c

    pltpu.sync_copy(src_hbm.at[wid], src_v)
    pltpu.sync_copy(dst_hbm.at[wid], dst_v)

    def zrow(i, _):
        zbuf[i, :] = jnp.zeros((DH,), jnp.float32)
        return 0
    lax.fori_loop(0, ZR, zrow, 0)

    def onesrow(i, _):
        ones_v[i, :] = jnp.ones((DH,), jnp.float32)
        return 0
    lax.fori_loop(0, B, onesrow, 0)

    def zcp(j, _):
        pltpu.sync_copy(zbuf, acc_sh.at[pl.ds(s * ZPS + j * ZR, ZR)])
        if with_deg:
            pltpu.sync_copy(zbuf, deg_sh.at[pl.ds(s * ZPS + j * ZR, ZR)])
        return 0
    lax.fori_loop(0, ZPS // ZR, zcp, 0)

    plsc.subcore_barrier()

    def issue(b, par):
        pltpu.async_copy(p_hbm.at[src_v.at[b]], rows[par], gsem[par])
        pltpu.async_copy(r_hbm.at[pl.ds(wid * EPW + b * B, B)], r_v[par],
                         rsem[par])

    def drain(par):
        pltpu.make_async_copy(p_hbm.at[pl.ds(0, B)], rows[par],
                              gsem[par]).wait()
        pltpu.make_async_copy(r_hbm.at[pl.ds(0, B)], r_v[par],
                              rsem[par]).wait()

    for par in range(NBUF):
        issue(par, par)

    def ring_body(jj, _):
        j0 = jj * NBUF
        for par in range(NBUF):
            b = j0 + par
            rbuf = rows[par]
            rcoef = r_v[par]
            drain(par)

            def group_body(g_, _g):
                le = lax.iota(jnp.int32, 16) + g_ * 16
                rk = [plsc.load_gather(rcoef, [le, _c16(k)]) for k in range(K)]
                acc = [None] * DH
                for k in [K] + list(range(K)):
                    for j in range(WPB):
                        w = plsc.load_gather(rbuf, [le, _c16(k * WPB + j)])
                        flo = plsc.bitcast(w << 16, jnp.float32)
                        fhi = plsc.bitcast(w & jnp.int32(-65536), jnp.float32)
                        if k == K:
                            acc[2 * j] = flo
                            acc[2 * j + 1] = fhi
                        else:
                            acc[2 * j] = acc[2 * j] + rk[k] * flo
                            acc[2 * j + 1] = acc[2 * j + 1] + rk[k] * fhi
                for o in range(DH):
                    plsc.store_scatter(msg_v, [le, _c16(o)], acc[o])
                return 0
            lax.fori_loop(0, GPB, group_body, 0)

            pltpu.sync_copy(msg_v, acc_sh.at[dst_v.at[b]], add=True)
            if with_deg:
                pltpu.sync_copy(ones_v, deg_sh.at[dst_v.at[b]], add=True)

            @pl.when(b + NBUF < NB)
            def _():
                issue(b + NBUF, par)
        return 0
    lax.fori_loop(0, NB // NBUF, ring_body, 0)

    plsc.subcore_barrier()

    pltpu.sync_copy(acc_sh.at[pl.ds(s * DS, DS)],
                    acc_out.at[c, pl.ds(s * DS, DS)])
    if with_deg:
        pltpu.sync_copy(deg_sh.at[pl.ds(s * DS, DS)],
                        deg_out.at[c, pl.ds(s * DS, DS)])

    @pl.when(s == NS - 1)
    def _():
        pltpu.sync_copy(acc_sh.at[pl.ds(DOFF, DREM)],
                        acc_out.at[c, pl.ds(DOFF, DREM)])
        if with_deg:
            pltpu.sync_copy(deg_sh.at[pl.ds(DOFF, DREM)],
                            deg_out.at[c, pl.ds(DOFF, DREM)])


def _edge_pass(p_tab, r_tab, srcr, dstr, with_deg):
    out_type = [jax.ShapeDtypeStruct((NC, N, DH), jnp.float32)]
    if with_deg:
        out_type.append(jax.ShapeDtypeStruct((NC, N, DH), jnp.float32))
    mesh = plsc.VectorSubcoreMesh(core_axis_name="c", subcore_axis_name="s")
    fn = pl.kernel(
        functools.partial(_edge_pass_body, with_deg),
        out_type=out_type,
        mesh=mesh,
        compiler_params=pltpu.CompilerParams(needs_layout_passes=False,
                                             use_tc_tiling_on_sc=False),
        scratch_types=(
            [pltpu.VMEM((NB, B), jnp.int32)] * 2
            + [pltpu.VMEM((B, KP), jnp.float32)] * NBUF
            + [pltpu.VMEM((B, PB), jnp.int32)] * NBUF
            + [pltpu.VMEM((B, DH), jnp.float32)] * 2
            + [pltpu.VMEM((ZR, DH), jnp.float32)]
            + [pltpu.VMEM_SHARED((NTRASH, DH), jnp.float32)] * 2
            + [pltpu.SemaphoreType.DMA] * (2 * NBUF)
        ),
    )
    return fn(p_tab, r_tab, srcr, dstr)


def _combine_body(a0_ref, a1_ref, d0_ref, d1_ref, xr_ref, bias_ref,
                  wb_ref, root_ref, p_ref, hr_ref):
    deg = jnp.maximum(d0_ref[...] + d1_ref[...], 1.0)
    agg = (a0_ref[...] + a1_ref[...]) / deg
    z = agg + xr_ref[...] + bias_ref[...]
    h = jnp.where(z > 0, z, jnp.exp(jnp.minimum(z, 0.0)) - 1.0)
    p_ref[...] = jnp.dot(h, wb_ref[...],
                         preferred_element_type=jnp.float32).astype(jnp.bfloat16)
    hr_ref[...] = jnp.dot(h, root_ref[...], preferred_element_type=jnp.float32)


def _combine(a0, a1, d0, d1, xr, bias, wbig, root):
    return pl.pallas_call(
        _combine_body,
        grid=(N // BN,),
        in_specs=[
            pl.BlockSpec((BN, DH), lambda i: (i, 0)),
            pl.BlockSpec((BN, DH), lambda i: (i, 0)),
            pl.BlockSpec((BN, DH), lambda i: (i, 0)),
            pl.BlockSpec((BN, DH), lambda i: (i, 0)),
            pl.BlockSpec((BN, DH), lambda i: (i, 0)),
            pl.BlockSpec((1, DH), lambda i: (0, 0)),
            pl.BlockSpec((DH, PW), lambda i: (0, 0)),
            pl.BlockSpec((DH, DH), lambda i: (0, 0)),
        ],
        out_specs=[
            pl.BlockSpec((BN, PW), lambda i: (i, 0)),
            pl.BlockSpec((BN, DH), lambda i: (i, 0)),
        ],
        out_shape=[
            jax.ShapeDtypeStruct((N, PW), jnp.bfloat16),
            jax.ShapeDtypeStruct((N, DH), jnp.float32),
        ],
    )(a0, a1, d0, d1, xr, bias, wbig, root)


def _final_body(a0_ref, a1_ref, d0_ref, d1_ref, hr_ref, bias_ref, batch_ref,
                fcw_ref, fcb_ref, psum_ref, csum_ref, out_ref):
    step = pl.program_id(0)

    @pl.when(step == 0)
    def _():
        psum_ref[...] = jnp.zeros_like(psum_ref)
        csum_ref[...] = jnp.zeros_like(csum_ref)

    deg = jnp.maximum(d0_ref[...] + d1_ref[...], 1.0)
    agg = (a0_ref[...] + a1_ref[...]) / deg
    z = agg + hr_ref[...] + bias_ref[...]
    h = jnp.where(z > 0, z, jnp.exp(jnp.minimum(z, 0.0)) - 1.0)

    row = batch_ref[...].reshape(1, BN)
    oh = (lax.broadcasted_iota(jnp.int32, (G, BN), 0)
          == jnp.broadcast_to(row, (G, BN))).astype(jnp.float32)
    psum_ref[...] += jnp.dot(oh, h, preferred_element_type=jnp.float32)
    csum_ref[...] += jnp.sum(oh, axis=1, keepdims=True) * jnp.ones(
        (1, DH), jnp.float32)

    @pl.when(step == (N // BN) - 1)
    def _():
        pooled = psum_ref[...] / jnp.maximum(csum_ref[...], 1.0)
        out_ref[...] = jnp.dot(pooled, fcw_ref[...],
                               preferred_element_type=jnp.float32) + fcb_ref[...]


def _final(a0, a1, d0, d1, hr, bias, batch3, fcw, fcb):
    outs = pl.pallas_call(
        _final_body,
        grid=(N // BN,),
        in_specs=[
            pl.BlockSpec((BN, DH), lambda i: (i, 0)),
            pl.BlockSpec((BN, DH), lambda i: (i, 0)),
            pl.BlockSpec((BN, DH), lambda i: (i, 0)),
            pl.BlockSpec((BN, DH), lambda i: (i, 0)),
            pl.BlockSpec((BN, DH), lambda i: (i, 0)),
            pl.BlockSpec((1, DH), lambda i: (0, 0)),
            pl.BlockSpec((1, 1, BN), lambda i: (i, 0, 0)),
            pl.BlockSpec((DH, DOUT), lambda i: (0, 0)),
            pl.BlockSpec((1, DOUT), lambda i: (0, 0)),
        ],
        out_specs=[
            pl.BlockSpec((G, DH), lambda i: (0, 0)),
            pl.BlockSpec((G, DH), lambda i: (0, 0)),
            pl.BlockSpec((G, DOUT), lambda i: (0, 0)),
        ],
        out_shape=[
            jax.ShapeDtypeStruct((G, DH), jnp.float32),
            jax.ShapeDtypeStruct((G, DH), jnp.float32),
            jax.ShapeDtypeStruct((G, DOUT), jnp.float32),
        ],
    )(a0, a1, d0, d1, hr, bias, batch3, fcw, fcb)
    return outs[2]


def _big_w(w2, b2, cin):
    a = w2.reshape(K, cin, DH).transpose(1, 0, 2).reshape(cin, K * DH)
    return jnp.concatenate([a, b2.reshape(cin, DH)], axis=1)


def _pad_mlp_params(w1, b1, g, be):
    s = jnp.float32(1.0) / jnp.sqrt(jnp.float32(1.0 + EPS))
    ge = g * s
    wa = jnp.pad(w1 * ge, ((0, 0), (0, KP - K)))
    ba = jnp.pad((b1 * ge + be).reshape(1, K), ((0, 0), (0, KP - K)))
    return wa, ba


def kernel(x, edge_index, edge_attr, batch, n1_W1, n1_b1, n1_g, n1_be, n1_W2,
           n1_b2, root1, bias1, n2_W1, n2_b1, n2_g, n2_be, n2_W2, n2_b2,
           root2, bias2, fc_W, fc_b):
    w1a, b1a = _pad_mlp_params(n1_W1, n1_b1, n1_g, n1_be)
    w1b, b1b = _pad_mlp_params(n2_W1, n2_b1, n2_g, n2_be)

    wb1 = _big_w(n1_W2, n1_b2, DIN)
    wb2 = _big_w(n2_W2, n2_b2, DH)

    eap = jnp.pad(edge_attr, ((0, EP - E), (0, 0)))
    srcr = jnp.pad(edge_index[0], (0, EP - E)).reshape(NW, NB, B)
    dstr = jnp.pad(edge_index[1], (0, EP - E),
                   constant_values=N).reshape(NW, NB, B)
    batch3 = batch.reshape(N // BN, 1, BN)

    def _pack(p):
        return jax.lax.bitcast_convert_type(p.reshape(N, PB, 2), jnp.int32)

    r1, r2 = _edge_mlp(eap, w1a, b1a, w1b, b1b)
    p1, xr1 = _node_table(x, wb1, root1)
    acc1, deg = _edge_pass(_pack(p1), r1, srcr, dstr, with_deg=True)
    p2, hr2 = _combine(acc1[0], acc1[1], deg[0], deg[1], xr1,
                       bias1.reshape(1, DH), wb2, root2)
    (acc2,) = _edge_pass(_pack(p2), r2, srcr, dstr, with_deg=False)
    out = _final(acc2[0], acc2[1], deg[0], deg[1], hr2,
                 bias2.reshape(1, DH), batch3, fc_W, fc_b.reshape(1, DOUT))
    return out


# speedup vs baseline: 2.1197x; 1.0067x over previous
import functools

import jax
import jax.numpy as jnp
import numpy as np
from jax import lax
from jax.experimental import pallas as pl
from jax.experimental.pallas import tpu as pltpu
from jax.experimental.pallas import tpu_sc as plsc

N = 10000
E = 160000
DIN = 128
DH = 16
DOUT = 10
G = 8
EPS = 1e-5
K = 25
KP = 32
PW = (K + 1) * DH
PB = PW // 2
WPB = DH // 2

NC = 2
NS = 16
NW = NC * NS
EP = 163840
EPW = EP // NW
B = 80
NB = EPW // B
GPB = B // 16
NBUF = 4
ZPS = 640
ZR = 64
NTRASH = NS * ZPS
DS = 624
DOFF = NS * DS
DREM = N - DOFF

BE = 4096
BN = 2000


def _c16(v):
    return jnp.full((16,), v, jnp.int32)


def _edge_mlp_body(ea_ref, w1_ref, b1_ref, w2_ref, b2_ref, r1_ref, r2_ref):
    ea = ea_ref[...]
    for w_ref, b_ref, out_ref in ((w1_ref, b1_ref, r1_ref), (w2_ref, b2_ref, r2_ref)):
        w = w_ref[...]
        acc = b_ref[...]
        acc = acc + ea[:, 0:1] * w[0:1, :]
        acc = acc + ea[:, 1:2] * w[1:2, :]
        acc = acc + ea[:, 2:3] * w[2:3, :]
        out_ref[...] = jnp.maximum(acc, 0.0)


def _edge_mlp(ea, w1a, b1a, w1b, b1b):
    return pl.pallas_call(
        _edge_mlp_body,
        grid=(EP // BE,),
        in_specs=[
            pl.BlockSpec((BE, 3), lambda i: (i, 0)),
            pl.BlockSpec((3, KP), lambda i: (0, 0)),
            pl.BlockSpec((1, KP), lambda i: (0, 0)),
            pl.BlockSpec((3, KP), lambda i: (0, 0)),
            pl.BlockSpec((1, KP), lambda i: (0, 0)),
        ],
        out_specs=[
            pl.BlockSpec((BE, KP), lambda i: (i, 0)),
            pl.BlockSpec((BE, KP), lambda i: (i, 0)),
        ],
        out_shape=[
            jax.ShapeDtypeStruct((EP, KP), jnp.float32),
            jax.ShapeDtypeStruct((EP, KP), jnp.float32),
        ],
    )(ea, w1a, b1a, w1b, b1b)


def _node_table_body(x_ref, wb_ref, root_ref, p_ref, xr_ref):
    x = x_ref[...]
    p_ref[...] = jnp.dot(x, wb_ref[...],
                         preferred_element_type=jnp.float32).astype(jnp.bfloat16)
    xr_ref[...] = jnp.dot(x, root_ref[...], preferred_element_type=jnp.float32)


def _node_table(x, wbig, root):
    cin = x.shape[1]
    return pl.pallas_call(
        _node_table_body,
        grid=(N // BN,),
        in_specs=[
            pl.BlockSpec((BN, cin), lambda i: (i, 0)),
            pl.BlockSpec((cin, PW), lambda i: (0, 0)),
            pl.BlockSpec((cin, DH), lambda i: (0, 0)),
        ],
        out_specs=[
            pl.BlockSpec((BN, PW), lambda i: (i, 0)),
            pl.BlockSpec((BN, DH), lambda i: (i, 0)),
        ],
        out_shape=[
            jax.ShapeDtypeStruct((N, PW), jnp.bfloat16),
            jax.ShapeDtypeStruct((N, DH), jnp.float32),
        ],
    )(x, wbig, root)


def _edge_pass_body(with_deg, p_hbm, r_hbm, src_hbm, dst_hbm, *rest):
    if with_deg:
        acc_out, deg_out = rest[0], rest[1]
        scratch = rest[2:]
    else:
        acc_out = rest[0]
        deg_out = None
        scratch = rest[1:]
    src_v, dst_v = scratch[0], scratch[1]
    r_v = scratch[2:2 + NBUF]
    rows = scratch[2 + NBUF:2 + 2 * NBUF]
    msg_v, ones_v, zbuf = scratch[2 + 2 * NBUF:5 + 2 * NBUF]
    acc_sh, deg_sh = scratch[5 + 2 * NBUF], scratch[6 + 2 * NBUF]
    gsem = scratch[7 + 2 * NBUF:7 + 3 * NBUF]
    rsem = scratch[7 + 3 * NBUF:7 + 4 * NBUF]

    c = lax.axis_index("c")
    s = lax.axis_index("s")
    wid = s * NC + c

    pltpu.sync_copy(src_hbm.at[wid], src_v)
    pltpu.sync_copy(dst_hbm.at[wid], dst_v)

    def zrow(i, _):
        zbuf[i, :] = jnp.zeros((DH,), jnp.float32)
        return 0
    lax.fori_loop(0, ZR, zrow, 0)

    def onesrow(i, _):
        ones_v[i, :] = jnp.ones((DH,), jnp.float32)
        return 0
    lax.fori_loop(0, B, onesrow, 0)

    def zcp(j, _):
        pltpu.sync_copy(zbuf, acc_sh.at[pl.ds(s * ZPS + j * ZR, ZR)])
        if with_deg:
            pltpu.sync_copy(zbuf, deg_sh.at[pl.ds(s * ZPS + j * ZR, ZR)])
        return 0
    lax.fori_loop(0, ZPS // ZR, zcp, 0)

    plsc.subcore_barrier()

    def issue(b, par):
        pltpu.async_copy(p_hbm.at[src_v.at[b]], rows[par], gsem[par])
        pltpu.async_copy(r_hbm.at[pl.ds(wid * EPW + b * B, B)], r_v[par],
                         rsem[par])

    def drain(par):
        pltpu.make_async_copy(p_hbm.at[pl.ds(0, B)], rows[par],
                              gsem[par]).wait()
        pltpu.make_async_copy(r_hbm.at[pl.ds(0, B)], r_v[par],
                              rsem[par]).wait()

    for par in range(NBUF):
        issue(par, par)

    def ring_body(jj, _):
        j0 = jj * NBUF
        for par in range(NBUF):
            b = j0 + par
            rbuf = rows[par]
            rcoef = r_v[par]
            drain(par)

            def group_body(g_, _g):
                le = lax.iota(jnp.int32, 16) + g_ * 16
                rk = [plsc.load_gather(rcoef, [le, _c16(k)]) for k in range(K)]
                acc = [None] * DH
                for k in [K] + list(range(K)):
                    for j in range(WPB):
                        w = plsc.load_gather(rbuf, [le, _c16(k * WPB + j)])
                        flo = plsc.bitcast(w << 16, jnp.float32)
                        fhi = plsc.bitcast(w, jnp.float32)
                        if k == K:
                            acc[2 * j] = flo
                            acc[2 * j + 1] = fhi
                        else:
                            acc[2 * j] = acc[2 * j] + rk[k] * flo
                            acc[2 * j + 1] = acc[2 * j + 1] + rk[k] * fhi
                for o in range(DH):
                    plsc.store_scatter(msg_v, [le, _c16(o)], acc[o])
                return 0
            lax.fori_loop(0, GPB, group_body, 0)

            pltpu.sync_copy(msg_v, acc_sh.at[dst_v.at[b]], add=True)
            if with_deg:
                pltpu.sync_copy(ones_v, deg_sh.at[dst_v.at[b]], add=True)

            @pl.when(b + NBUF < NB)
            def _():
                issue(b + NBUF, par)
        return 0
    lax.fori_loop(0, NB // NBUF, ring_body, 0)

    plsc.subcore_barrier()

    pltpu.sync_copy(acc_sh.at[pl.ds(s * DS, DS)],
                    acc_out.at[c, pl.ds(s * DS, DS)])
    if with_deg:
        pltpu.sync_copy(deg_sh.at[pl.ds(s * DS, DS)],
                        deg_out.at[c, pl.ds(s * DS, DS)])

    @pl.when(s == NS - 1)
    def _():
        pltpu.sync_copy(acc_sh.at[pl.ds(DOFF, DREM)],
                        acc_out.at[c, pl.ds(DOFF, DREM)])
        if with_deg:
            pltpu.sync_copy(deg_sh.at[pl.ds(DOFF, DREM)],
                            deg_out.at[c, pl.ds(DOFF, DREM)])


def _edge_pass(p_tab, r_tab, srcr, dstr, with_deg):
    out_type = [jax.ShapeDtypeStruct((NC, N, DH), jnp.float32)]
    if with_deg:
        out_type.append(jax.ShapeDtypeStruct((NC, N, DH), jnp.float32))
    mesh = plsc.VectorSubcoreMesh(core_axis_name="c", subcore_axis_name="s")
    fn = pl.kernel(
        functools.partial(_edge_pass_body, with_deg),
        out_type=out_type,
        mesh=mesh,
        compiler_params=pltpu.CompilerParams(needs_layout_passes=False,
                                             use_tc_tiling_on_sc=False),
        scratch_types=(
            [pltpu.VMEM((NB, B), jnp.int32)] * 2
            + [pltpu.VMEM((B, KP), jnp.float32)] * NBUF
            + [pltpu.VMEM((B, PB), jnp.int32)] * NBUF
            + [pltpu.VMEM((B, DH), jnp.float32)] * 2
            + [pltpu.VMEM((ZR, DH), jnp.float32)]
            + [pltpu.VMEM_SHARED((NTRASH, DH), jnp.float32)] * 2
            + [pltpu.SemaphoreType.DMA] * (2 * NBUF)
        ),
    )
    return fn(p_tab, r_tab, srcr, dstr)


def _combine_body(a0_ref, a1_ref, d0_ref, d1_ref, xr_ref, bias_ref,
                  wb_ref, root_ref, p_ref, hr_ref):
    deg = jnp.maximum(d0_ref[...] + d1_ref[...], 1.0)
    agg = (a0_ref[...] + a1_ref[...]) / deg
    z = agg + xr_ref[...] + bias_ref[...]
    h = jnp.where(z > 0, z, jnp.exp(jnp.minimum(z, 0.0)) - 1.0)
    p_ref[...] = jnp.dot(h, wb_ref[...],
                         preferred_element_type=jnp.float32).astype(jnp.bfloat16)
    hr_ref[...] = jnp.dot(h, root_ref[...], preferred_element_type=jnp.float32)


def _combine(a0, a1, d0, d1, xr, bias, wbig, root):
    return pl.pallas_call(
        _combine_body,
        grid=(N // BN,),
        in_specs=[
            pl.BlockSpec((BN, DH), lambda i: (i, 0)),
            pl.BlockSpec((BN, DH), lambda i: (i, 0)),
            pl.BlockSpec((BN, DH), lambda i: (i, 0)),
            pl.BlockSpec((BN, DH), lambda i: (i, 0)),
            pl.BlockSpec((BN, DH), lambda i: (i, 0)),
            pl.BlockSpec((1, DH), lambda i: (0, 0)),
            pl.BlockSpec((DH, PW), lambda i: (0, 0)),
            pl.BlockSpec((DH, DH), lambda i: (0, 0)),
        ],
        out_specs=[
            pl.BlockSpec((BN, PW), lambda i: (i, 0)),
            pl.BlockSpec((BN, DH), lambda i: (i, 0)),
        ],
        out_shape=[
            jax.ShapeDtypeStruct((N, PW), jnp.bfloat16),
            jax.ShapeDtypeStruct((N, DH), jnp.float32),
        ],
    )(a0, a1, d0, d1, xr, bias, wbig, root)


def _final_body(a0_ref, a1_ref, d0_ref, d1_ref, hr_ref, bias_ref, batch_ref,
                fcw_ref, fcb_ref, psum_ref, csum_ref, out_ref):
    step = pl.program_id(0)

    @pl.when(step == 0)
    def _():
        psum_ref[...] = jnp.zeros_like(psum_ref)
        csum_ref[...] = jnp.zeros_like(csum_ref)

    deg = jnp.maximum(d0_ref[...] + d1_ref[...], 1.0)
    agg = (a0_ref[...] + a1_ref[...]) / deg
    z = agg + hr_ref[...] + bias_ref[...]
    h = jnp.where(z > 0, z, jnp.exp(jnp.minimum(z, 0.0)) - 1.0)

    row = batch_ref[...].reshape(1, BN)
    oh = (lax.broadcasted_iota(jnp.int32, (G, BN), 0)
          == jnp.broadcast_to(row, (G, BN))).astype(jnp.float32)
    psum_ref[...] += jnp.dot(oh, h, preferred_element_type=jnp.float32)
    csum_ref[...] += jnp.sum(oh, axis=1, keepdims=True) * jnp.ones(
        (1, DH), jnp.float32)

    @pl.when(step == (N // BN) - 1)
    def _():
        pooled = psum_ref[...] / jnp.maximum(csum_ref[...], 1.0)
        out_ref[...] = jnp.dot(pooled, fcw_ref[...],
                               preferred_element_type=jnp.float32) + fcb_ref[...]


def _final(a0, a1, d0, d1, hr, bias, batch3, fcw, fcb):
    outs = pl.pallas_call(
        _final_body,
        grid=(N // BN,),
        in_specs=[
            pl.BlockSpec((BN, DH), lambda i: (i, 0)),
            pl.BlockSpec((BN, DH), lambda i: (i, 0)),
            pl.BlockSpec((BN, DH), lambda i: (i, 0)),
            pl.BlockSpec((BN, DH), lambda i: (i, 0)),
            pl.BlockSpec((BN, DH), lambda i: (i, 0)),
            pl.BlockSpec((1, DH), lambda i: (0, 0)),
            pl.BlockSpec((1, 1, BN), lambda i: (i, 0, 0)),
            pl.BlockSpec((DH, DOUT), lambda i: (0, 0)),
            pl.BlockSpec((1, DOUT), lambda i: (0, 0)),
        ],
        out_specs=[
            pl.BlockSpec((G, DH), lambda i: (0, 0)),
            pl.BlockSpec((G, DH), lambda i: (0, 0)),
            pl.BlockSpec((G, DOUT), lambda i: (0, 0)),
        ],
        out_shape=[
            jax.ShapeDtypeStruct((G, DH), jnp.float32),
            jax.ShapeDtypeStruct((G, DH), jnp.float32),
            jax.ShapeDtypeStruct((G, DOUT), jnp.float32),
        ],
    )(a0, a1, d0, d1, hr, bias, batch3, fcw, fcb)
    return outs[2]


def _big_w(w2, b2, cin):
    a = w2.reshape(K, cin, DH).transpose(1, 0, 2).reshape(cin, K * DH)
    return jnp.concatenate([a, b2.reshape(cin, DH)], axis=1)


def _pad_mlp_params(w1, b1, g, be):
    s = jnp.float32(1.0) / jnp.sqrt(jnp.float32(1.0 + EPS))
    ge = g * s
    wa = jnp.pad(w1 * ge, ((0, 0), (0, KP - K)))
    ba = jnp.pad((b1 * ge + be).reshape(1, K), ((0, 0), (0, KP - K)))
    return wa, ba


def kernel(x, edge_index, edge_attr, batch, n1_W1, n1_b1, n1_g, n1_be, n1_W2,
           n1_b2, root1, bias1, n2_W1, n2_b1, n2_g, n2_be, n2_W2, n2_b2,
           root2, bias2, fc_W, fc_b):
    w1a, b1a = _pad_mlp_params(n1_W1, n1_b1, n1_g, n1_be)
    w1b, b1b = _pad_mlp_params(n2_W1, n2_b1, n2_g, n2_be)

    wb1 = _big_w(n1_W2, n1_b2, DIN)
    wb2 = _big_w(n2_W2, n2_b2, DH)

    eap = jnp.pad(edge_attr, ((0, EP - E), (0, 0)))
    srcr = jnp.pad(edge_index[0], (0, EP - E)).reshape(NW, NB, B)
    dstr = jnp.pad(edge_index[1], (0, EP - E),
                   constant_values=N).reshape(NW, NB, B)
    batch3 = batch.reshape(N // BN, 1, BN)

    def _pack(p):
        return jax.lax.bitcast_convert_type(p.reshape(N, PB, 2), jnp.int32)

    r1, r2 = _edge_mlp(eap, w1a, b1a, w1b, b1b)
    p1, xr1 = _node_table(x, wb1, root1)
    acc1, deg = _edge_pass(_pack(p1), r1, srcr, dstr, with_deg=True)
    p2, hr2 = _combine(acc1[0], acc1[1], deg[0], deg[1], xr1,
                       bias1.reshape(1, DH), wb2, root2)
    (acc2,) = _edge_pass(_pack(p2), r2, srcr, dstr, with_deg=False)
    out = _final(acc2[0], acc2[1], deg[0], deg[1], hr2,
                 bias2.reshape(1, DH), batch3, fc_W, fc_b.reshape(1, DOUT))
    return out
